# baseline (speedup 1.0000x reference)
import jax
import jax.numpy as jnp
import numpy as np
from jax import lax
from jax.experimental import pallas as pl
from jax.experimental.pallas import tpu as pltpu

_NB = 64
_SIGMA = 0.1 * (1.0 / _NB)
_EPS = float(np.finfo(np.float32).eps)
_K2 = float(np.log2(np.e) / (_SIGMA * _SIGMA))


def _minmax_body(x1_ref, x2_ref, o_ref):
    o_ref[0] = jnp.min(x1_ref[...])
    o_ref[1] = jnp.max(x1_ref[...])
    o_ref[2] = jnp.min(x2_ref[...])
    o_ref[3] = jnp.max(x2_ref[...])


def _hist_body(mm_ref, x1_ref, x2_ref, o_ref, acc_ref):
    j = pl.program_id(1)
    mn1 = mm_ref[0]
    mx1 = mm_ref[1]
    mn2 = mm_ref[2]
    mx2 = mm_ref[3]
    inv1 = 1.0 / (mx1 - mn1 + _EPS)
    inv2 = 1.0 / (mx2 - mn2 + _EPS)

    a = x1_ref[0] * (0.5 * inv1)
    b = x2_ref[0] * (0.5 * inv2)
    z = a + b - 0.5 * (mn1 * inv1 + mn2 * inv2)
    r = (a - b) * 2.0 + (mn2 * inv2 - mn1 * inv1)
    h = (r * r) * (-0.25 * _K2)

    ci = lax.broadcasted_iota(jnp.int32, (_NB, 1, 1), 0)
    c = (ci.astype(jnp.float32) + 0.5) * (1.0 / _NB)
    zc = z[None] - c
    e = (zc * zc) * (-_K2) + h[None]
    w = jnp.exp2(e)

    @pl.when(j == 0)
    def _():
        acc_ref[...] = w

    @pl.when(j > 0)
    def _():
        acc_ref[...] = acc_ref[...] + w

    @pl.when(j == pl.num_programs(1) - 1)
    def _():
        t = jnp.sum(acc_ref[...], axis=1)
        o_ref[0] = (t[:, 0:128] + t[:, 128:256]
                    + t[:, 256:384] + t[:, 384:512])


def _mi_body(p_ref, o_ref):
    nc = p_ref.shape[0]
    total = 4 * 1 * 512 * 512
    hist = jnp.sum(p_ref[...], axis=-1)
    hist = hist * (1.0 / total)
    s = jnp.sum(hist, axis=-1, keepdims=True)
    hist = hist / (s + _EPS)
    px = jnp.sum(hist, axis=-1, keepdims=True)
    py = jnp.sum(hist, axis=0, keepdims=True)
    px_py = px * py
    mi = jnp.sum(hist * jnp.log((hist + _EPS) / (px_py + _EPS) + _EPS))
    o_ref[0] = mi


def kernel(img1, img2):
    n, c, h, w = img1.shape
    nc = n * c
    x1 = img1.reshape(nc, h, w)
    x2 = img2.reshape(nc, h, w)

    mm = pl.pallas_call(
        _minmax_body,
        out_shape=jax.ShapeDtypeStruct((4,), jnp.float32),
        out_specs=pl.BlockSpec(memory_space=pltpu.SMEM),
    )(x1, x2)

    rows = 8
    nchunks = h // rows
    part = pl.pallas_call(
        _hist_body,
        grid=(nc, nchunks),
        in_specs=[
            pl.BlockSpec(memory_space=pltpu.SMEM),
            pl.BlockSpec((1, rows, w), lambda i, j: (i, j, 0)),
            pl.BlockSpec((1, rows, w), lambda i, j: (i, j, 0)),
        ],
        out_specs=pl.BlockSpec((1, _NB, 128), lambda i, j: (i, 0, 0)),
        out_shape=jax.ShapeDtypeStruct((nc, _NB, 128), jnp.float32),
        scratch_shapes=[pltpu.VMEM((_NB, rows, w), jnp.float32)],
        compiler_params=pltpu.CompilerParams(
            dimension_semantics=("arbitrary", "arbitrary"),
        ),
    )(mm, x1, x2)

    mi = pl.pallas_call(
        _mi_body,
        out_shape=jax.ShapeDtypeStruct((1,), jnp.float32),
        out_specs=pl.BlockSpec(memory_space=pltpu.SMEM),
    )(part)
    return mi[0]


# speedup vs baseline: 1.7888x; 1.7888x over previous
import jax
import jax.numpy as jnp
import numpy as np
from jax import lax
from jax.experimental import pallas as pl
from jax.experimental.pallas import tpu as pltpu

_NB = 64
_SIGMA = 0.1 * (1.0 / _NB)
_EPS = float(np.finfo(np.float32).eps)
_K2 = float(np.log2(np.e) / (_SIGMA * _SIGMA))


def _minmax_body(x1_ref, x2_ref, o_ref):
    o_ref[0] = jnp.min(x1_ref[...])
    o_ref[1] = jnp.max(x1_ref[...])
    o_ref[2] = jnp.min(x2_ref[...])
    o_ref[3] = jnp.max(x2_ref[...])


_SQK = float(np.sqrt(_K2))


def _hist_body(mm_ref, x1_ref, x2_ref, o_ref, acc_ref):
    j = pl.program_id(1)
    mn1 = mm_ref[0]
    mx1 = mm_ref[1]
    mn2 = mm_ref[2]
    mx2 = mm_ref[3]
    inv1 = 1.0 / (mx1 - mn1 + _EPS)
    inv2 = 1.0 / (mx2 - mn2 + _EPS)

    a = x1_ref[0] * (0.5 * inv1)
    b = x2_ref[0] * (0.5 * inv2)
    z = a + b - 0.5 * (mn1 * inv1 + mn2 * inv2)
    r = (a - b) * 2.0 + (mn2 * inv2 - mn1 * inv1)
    h = (r * r) * (-0.25 * _K2)
    zs = z * _SQK

    @pl.when(j == 0)
    def _():
        acc_ref[...] = jnp.zeros_like(acc_ref)

    for bi in range(_NB):
        cbs = _SQK * ((bi + 0.5) / _NB)
        zc = zs - cbs
        e = h - zc * zc
        acc_ref[bi] = acc_ref[bi] + jnp.exp2(e)

    @pl.when(j == pl.num_programs(1) - 1)
    def _():
        acc = acc_ref[...]
        u = (acc[:, :, 0:128] + acc[:, :, 128:256]
             + acc[:, :, 256:384] + acc[:, :, 384:512])
        o_ref[0] = jnp.sum(u, axis=1)


def _mi_body(p_ref, o_ref):
    nc = p_ref.shape[0]
    total = 4 * 1 * 512 * 512
    hist = jnp.sum(p_ref[...], axis=-1)
    hist = hist * (1.0 / total)
    s = jnp.sum(hist, axis=-1, keepdims=True)
    hist = hist / (s + _EPS)
    px = jnp.sum(hist, axis=-1, keepdims=True)
    py = jnp.sum(hist, axis=0, keepdims=True)
    px_py = px * py
    mi = jnp.sum(hist * jnp.log((hist + _EPS) / (px_py + _EPS) + _EPS))
    o_ref[0] = mi


def kernel(img1, img2):
    n, c, h, w = img1.shape
    nc = n * c
    x1 = img1.reshape(nc, h, w)
    x2 = img2.reshape(nc, h, w)

    mm = pl.pallas_call(
        _minmax_body,
        out_shape=jax.ShapeDtypeStruct((4,), jnp.float32),
        out_specs=pl.BlockSpec(memory_space=pltpu.SMEM),
    )(x1, x2)

    rows = 16
    nchunks = h // rows
    part = pl.pallas_call(
        _hist_body,
        grid=(nc, nchunks),
        in_specs=[
            pl.BlockSpec(memory_space=pltpu.SMEM),
            pl.BlockSpec((1, rows, w), lambda i, j: (i, j, 0)),
            pl.BlockSpec((1, rows, w), lambda i, j: (i, j, 0)),
        ],
        out_specs=pl.BlockSpec((1, _NB, 128), lambda i, j: (i, 0, 0)),
        out_shape=jax.ShapeDtypeStruct((nc, _NB, 128), jnp.float32),
        scratch_shapes=[pltpu.VMEM((_NB, rows, w), jnp.float32)],
        compiler_params=pltpu.CompilerParams(
            dimension_semantics=("arbitrary", "arbitrary"),
        ),
    )(mm, x1, x2)

    mi = pl.pallas_call(
        _mi_body,
        out_shape=jax.ShapeDtypeStruct((1,), jnp.float32),
        out_specs=pl.BlockSpec(memory_space=pltpu.SMEM),
    )(part)
    return mi[0]


# speedup vs baseline: 3.6549x; 2.0432x over previous
import jax
import jax.numpy as jnp
import numpy as np
from jax.experimental import pallas as pl
from jax.experimental.pallas import tpu as pltpu

_NB = 64
_SIGMA = 0.1 * (1.0 / _NB)
_EPS = float(np.finfo(np.float32).eps)
_K2 = float(np.log2(np.e) / (_SIGMA * _SIGMA))
_SQK = float(np.sqrt(_K2))


def _body(x1_ref, x2_ref, mi_ref, mm_ref, hist_ref):
    p = pl.program_id(0)
    i = pl.program_id(1)
    nc = pl.num_programs(1)

    @pl.when(p == 0)
    def _():
        mn1 = jnp.min(x1_ref[...])
        mx1 = jnp.max(x1_ref[...])
        mn2 = jnp.min(x2_ref[...])
        mx2 = jnp.max(x2_ref[...])

        @pl.when(i == 0)
        def _():
            mm_ref[0] = mn1
            mm_ref[1] = mx1
            mm_ref[2] = mn2
            mm_ref[3] = mx2

        @pl.when(i > 0)
        def _():
            mm_ref[0] = jnp.minimum(mm_ref[0], mn1)
            mm_ref[1] = jnp.maximum(mm_ref[1], mx1)
            mm_ref[2] = jnp.minimum(mm_ref[2], mn2)
            mm_ref[3] = jnp.maximum(mm_ref[3], mx2)

    @pl.when(p == 1)
    def _():
        mn1 = mm_ref[0]
        mx1 = mm_ref[1]
        mn2 = mm_ref[2]
        mx2 = mm_ref[3]
        inv1 = 1.0 / (mx1 - mn1 + _EPS)
        inv2 = 1.0 / (mx2 - mn2 + _EPS)

        hist_ref[i] = jnp.zeros_like(hist_ref[i])

        rows = x1_ref.shape[1]
        dlt = _SQK / _NB
        for s in range(0, rows, 8):
            a = x1_ref[0, s:s + 8] * (0.5 * inv1)
            b = x2_ref[0, s:s + 8] * (0.5 * inv2)
            z = a + b - 0.5 * (mn1 * inv1 + mn2 * inv2)
            r = (a - b) * 2.0 + (mn2 * inv2 - mn1 * inv1)
            h = (r * r) * (-0.25 * _K2)
            zs = z * _SQK
            e = None
            d = None
            for bi in range(_NB):
                if bi % 8 == 0:
                    cbs = _SQK * ((bi + 0.5) / _NB)
                    zc = zs - cbs
                    e = h - zc * zc
                    d = zc * (2.0 * dlt) - dlt * dlt
                else:
                    e = e + d
                    d = d - 2.0 * dlt * dlt
                w = jnp.exp2(e)
                t = (w[:, 0:128] + w[:, 128:256]
                     + w[:, 256:384] + w[:, 384:512])
                hist_ref[i, bi] = hist_ref[i, bi] + t

        @pl.when(i == nc - 1)
        def _():
            total = 4 * 1 * 512 * 512
            part = jnp.sum(hist_ref[...], axis=2)
            hist = jnp.sum(part, axis=-1)
            hist = hist * (1.0 / total)
            ssum = jnp.sum(hist, axis=-1, keepdims=True)
            hist = hist / (ssum + _EPS)
            px = jnp.sum(hist, axis=-1, keepdims=True)
            py = jnp.sum(hist, axis=0, keepdims=True)
            px_py = px * py
            mi = jnp.sum(hist * jnp.log((hist + _EPS) / (px_py + _EPS) + _EPS))
            mi_ref[0] = mi


def kernel(img1, img2):
    n, c, h, w = img1.shape
    nc = n * c
    x1 = img1.reshape(nc, h, w)
    x2 = img2.reshape(nc, h, w)

    mi = pl.pallas_call(
        _body,
        grid=(2, nc),
        in_specs=[
            pl.BlockSpec((1, h, w), lambda p, i: (i, 0, 0)),
            pl.BlockSpec((1, h, w), lambda p, i: (i, 0, 0)),
        ],
        out_specs=pl.BlockSpec(memory_space=pltpu.SMEM),
        out_shape=jax.ShapeDtypeStruct((1,), jnp.float32),
        scratch_shapes=[
            pltpu.SMEM((8,), jnp.float32),
            pltpu.VMEM((nc, _NB, 8, 128), jnp.float32),
        ],
        compiler_params=pltpu.CompilerParams(
            dimension_semantics=("arbitrary", "arbitrary"),
        ),
    )(x1, x2)
    return mi[0]


# speedup vs baseline: 3.6611x; 1.0017x over previous
import jax
import jax.numpy as jnp
import numpy as np
from jax.experimental import pallas as pl
from jax.experimental.pallas import tpu as pltpu

_NB = 64
_SIGMA = 0.1 * (1.0 / _NB)
_EPS = float(np.finfo(np.float32).eps)
_K2 = float(np.log2(np.e) / (_SIGMA * _SIGMA))
_SQK = float(np.sqrt(_K2))


def _fold_extreme(v, op):
    t = v[0:8]
    for s in range(8, v.shape[0], 8):
        t = op(t, v[s:s + 8])
    return op(op(t[:, 0:128], t[:, 128:256]),
              op(t[:, 256:384], t[:, 384:512]))


def _body(x1_ref, x2_ref, mi_ref, mm_ref, hist_ref, mmv_ref):
    p = pl.program_id(0)
    i = pl.program_id(1)
    nc = pl.num_programs(1)

    @pl.when(p == 0)
    def _():
        mn1 = _fold_extreme(x1_ref[0], jnp.minimum)
        mx1 = _fold_extreme(x1_ref[0], jnp.maximum)
        mn2 = _fold_extreme(x2_ref[0], jnp.minimum)
        mx2 = _fold_extreme(x2_ref[0], jnp.maximum)

        @pl.when(i == 0)
        def _():
            mmv_ref[0] = mn1
            mmv_ref[1] = mx1
            mmv_ref[2] = mn2
            mmv_ref[3] = mx2

        @pl.when(i > 0)
        def _():
            mmv_ref[0] = jnp.minimum(mmv_ref[0], mn1)
            mmv_ref[1] = jnp.maximum(mmv_ref[1], mx1)
            mmv_ref[2] = jnp.minimum(mmv_ref[2], mn2)
            mmv_ref[3] = jnp.maximum(mmv_ref[3], mx2)

    @pl.when(p == 1)
    def _():
        @pl.when(i == 0)
        def _():
            mm_ref[0] = jnp.min(mmv_ref[0])
            mm_ref[1] = jnp.max(mmv_ref[1])
            mm_ref[2] = jnp.min(mmv_ref[2])
            mm_ref[3] = jnp.max(mmv_ref[3])

        mn1 = mm_ref[0]
        mx1 = mm_ref[1]
        mn2 = mm_ref[2]
        mx2 = mm_ref[3]
        inv1 = 1.0 / (mx1 - mn1 + _EPS)
        inv2 = 1.0 / (mx2 - mn2 + _EPS)

        hist_ref[i] = jnp.zeros_like(hist_ref[i])

        rows = x1_ref.shape[1]
        dlt = _SQK / _NB
        for s in range(0, rows, 8):
            a = x1_ref[0, s:s + 8] * (0.5 * inv1)
            b = x2_ref[0, s:s + 8] * (0.5 * inv2)
            z = a + b - 0.5 * (mn1 * inv1 + mn2 * inv2)
            r = (a - b) * 2.0 + (mn2 * inv2 - mn1 * inv1)
            h = (r * r) * (-0.25 * _K2)
            zs = z * _SQK
            e = None
            d = None
            for bi in range(_NB):
                if bi % 8 == 0:
                    cbs = _SQK * ((bi + 0.5) / _NB)
                    zc = zs - cbs
                    e = h - zc * zc
                    d = zc * (2.0 * dlt) - dlt * dlt
                else:
                    e = e + d
                    d = d - 2.0 * dlt * dlt
                w = jnp.exp2(e)
                t = (w[:, 0:128] + w[:, 128:256]
                     + w[:, 256:384] + w[:, 384:512])
                hist_ref[i, bi] = hist_ref[i, bi] + t

        @pl.when(i == nc - 1)
        def _():
            total = 4 * 1 * 512 * 512
            part = jnp.sum(hist_ref[...], axis=2)
            hist = jnp.sum(part, axis=-1)
            hist = hist * (1.0 / total)
            ssum = jnp.sum(hist, axis=-1, keepdims=True)
            hist = hist / (ssum + _EPS)
            px = jnp.sum(hist, axis=-1, keepdims=True)
            py = jnp.sum(hist, axis=0, keepdims=True)
            px_py = px * py
            mi = jnp.sum(hist * jnp.log((hist + _EPS) / (px_py + _EPS) + _EPS))
            mi_ref[0] = mi


def kernel(img1, img2):
    n, c, h, w = img1.shape
    nc = n * c
    x1 = img1.reshape(nc, h, w)
    x2 = img2.reshape(nc, h, w)

    mi = pl.pallas_call(
        _body,
        grid=(2, nc),
        in_specs=[
            pl.BlockSpec((1, h, w), lambda p, i: (i, 0, 0)),
            pl.BlockSpec((1, h, w), lambda p, i: (i, 0, 0)),
        ],
        out_specs=pl.BlockSpec(memory_space=pltpu.SMEM),
        out_shape=jax.ShapeDtypeStruct((1,), jnp.float32),
        scratch_shapes=[
            pltpu.SMEM((8,), jnp.float32),
            pltpu.VMEM((nc, _NB, 8, 128), jnp.float32),
            pltpu.VMEM((4, 8, 128), jnp.float32),
        ],
        compiler_params=pltpu.CompilerParams(
            dimension_semantics=("arbitrary", "arbitrary"),
        ),
    )(x1, x2)
    return mi[0]


# speedup vs baseline: 3.7573x; 1.0263x over previous
import jax
import jax.numpy as jnp
import numpy as np
from jax.experimental import pallas as pl
from jax.experimental.pallas import tpu as pltpu

_NB = 64
_SIGMA = 0.1 * (1.0 / _NB)
_EPS = float(np.finfo(np.float32).eps)
_K2 = float(np.log2(np.e) / (_SIGMA * _SIGMA))
_SQK = float(np.sqrt(_K2))


def _fold_extreme(v, op):
    t = v[0:8]
    for s in range(8, v.shape[0], 8):
        t = op(t, v[s:s + 8])
    return op(op(t[:, 0:128], t[:, 128:256]),
              op(t[:, 256:384], t[:, 384:512]))


def _body(x1_ref, x2_ref, mi_ref, mm_ref, hist_ref, mmv_ref):
    p = pl.program_id(0)
    i = pl.program_id(1)
    nc = pl.num_programs(1)

    @pl.when(p == 0)
    def _():
        mn1 = _fold_extreme(x1_ref[0], jnp.minimum)
        mx1 = _fold_extreme(x1_ref[0], jnp.maximum)
        mn2 = _fold_extreme(x2_ref[0], jnp.minimum)
        mx2 = _fold_extreme(x2_ref[0], jnp.maximum)

        @pl.when(i == 0)
        def _():
            mmv_ref[0] = mn1
            mmv_ref[1] = mx1
            mmv_ref[2] = mn2
            mmv_ref[3] = mx2

        @pl.when(i > 0)
        def _():
            mmv_ref[0] = jnp.minimum(mmv_ref[0], mn1)
            mmv_ref[1] = jnp.maximum(mmv_ref[1], mx1)
            mmv_ref[2] = jnp.minimum(mmv_ref[2], mn2)
            mmv_ref[3] = jnp.maximum(mmv_ref[3], mx2)

    @pl.when(p == 1)
    def _():
        @pl.when(i == 0)
        def _():
            mm_ref[0] = jnp.min(mmv_ref[0])
            mm_ref[1] = jnp.max(mmv_ref[1])
            mm_ref[2] = jnp.min(mmv_ref[2])
            mm_ref[3] = jnp.max(mmv_ref[3])

        mn1 = mm_ref[0]
        mx1 = mm_ref[1]
        mn2 = mm_ref[2]
        mx2 = mm_ref[3]
        inv1 = 1.0 / (mx1 - mn1 + _EPS)
        inv2 = 1.0 / (mx2 - mn2 + _EPS)

        rows = x1_ref.shape[1]
        dlt = _SQK / _NB
        for s in range(0, rows, 8):
            a = x1_ref[0, s:s + 8] * (0.5 * inv1)
            b = x2_ref[0, s:s + 8] * (0.5 * inv2)
            z = a + b - 0.5 * (mn1 * inv1 + mn2 * inv2)
            r = (a - b) * 2.0 + (mn2 * inv2 - mn1 * inv1)
            h = (r * r) * (-0.25 * _K2)
            zs = z * _SQK
            e = None
            d = None
            for bi in range(_NB):
                if bi % 16 == 0:
                    cbs = _SQK * ((bi + 0.5) / _NB)
                    zc = zs - cbs
                    e = h - zc * zc
                    d = zc * (2.0 * dlt) - dlt * dlt
                else:
                    e = e + d
                    d = d - 2.0 * dlt * dlt
                w = jnp.exp2(e)
                t = (w[:, 0:128] + w[:, 128:256]
                     + w[:, 256:384] + w[:, 384:512])
                if s == 0:
                    hist_ref[i, bi] = t
                else:
                    hist_ref[i, bi] = hist_ref[i, bi] + t

        @pl.when(i == nc - 1)
        def _():
            total = 4 * 1 * 512 * 512
            part = jnp.sum(hist_ref[...], axis=2)
            hist = jnp.sum(part, axis=-1)
            hist = hist * (1.0 / total)
            ssum = jnp.sum(hist, axis=-1, keepdims=True)
            hist = hist / (ssum + _EPS)
            px = jnp.sum(hist, axis=-1, keepdims=True)
            py = jnp.sum(hist, axis=0, keepdims=True)
            px_py = px * py
            mi = jnp.sum(hist * jnp.log((hist + _EPS) / (px_py + _EPS) + _EPS))
            mi_ref[0] = mi


def kernel(img1, img2):
    n, c, h, w = img1.shape
    nc = n * c
    x1 = img1.reshape(nc, h, w)
    x2 = img2.reshape(nc, h, w)

    mi = pl.pallas_call(
        _body,
        grid=(2, nc),
        in_specs=[
            pl.BlockSpec((1, h, w), lambda p, i: (i, 0, 0)),
            pl.BlockSpec((1, h, w), lambda p, i: (i, 0, 0)),
        ],
        out_specs=pl.BlockSpec(memory_space=pltpu.SMEM),
        out_shape=jax.ShapeDtypeStruct((1,), jnp.float32),
        scratch_shapes=[
            pltpu.SMEM((8,), jnp.float32),
            pltpu.VMEM((nc, _NB, 8, 128), jnp.float32),
            pltpu.VMEM((4, 8, 128), jnp.float32),
        ],
        compiler_params=pltpu.CompilerParams(
            dimension_semantics=("arbitrary", "arbitrary"),
        ),
    )(x1, x2)
    return mi[0]


# speedup vs baseline: 3.7620x; 1.0013x over previous
import jax
import jax.numpy as jnp
import numpy as np
from jax.experimental import pallas as pl
from jax.experimental.pallas import tpu as pltpu

_NB = 64
_SIGMA = 0.1 * (1.0 / _NB)
_EPS = float(np.finfo(np.float32).eps)
_K2 = float(np.log2(np.e) / (_SIGMA * _SIGMA))
_SQK = float(np.sqrt(_K2))


def _fold_extreme(v, op):
    t = v[0:8]
    for s in range(8, v.shape[0], 8):
        t = op(t, v[s:s + 8])
    return op(op(t[:, 0:128], t[:, 128:256]),
              op(t[:, 256:384], t[:, 384:512]))


def _body(x1_ref, x2_ref, mi_ref, mm_ref, hist_ref, mmv_ref):
    p = pl.program_id(0)
    i = pl.program_id(1)
    nc = pl.num_programs(1)

    @pl.when(p == 0)
    def _():
        mn1 = _fold_extreme(x1_ref[0], jnp.minimum)
        mx1 = _fold_extreme(x1_ref[0], jnp.maximum)
        mn2 = _fold_extreme(x2_ref[0], jnp.minimum)
        mx2 = _fold_extreme(x2_ref[0], jnp.maximum)

        @pl.when(i == 0)
        def _():
            mmv_ref[0] = mn1
            mmv_ref[1] = mx1
            mmv_ref[2] = mn2
            mmv_ref[3] = mx2

        @pl.when(i > 0)
        def _():
            mmv_ref[0] = jnp.minimum(mmv_ref[0], mn1)
            mmv_ref[1] = jnp.maximum(mmv_ref[1], mx1)
            mmv_ref[2] = jnp.minimum(mmv_ref[2], mn2)
            mmv_ref[3] = jnp.maximum(mmv_ref[3], mx2)

    @pl.when(p == 1)
    def _():
        @pl.when(i == 0)
        def _():
            mm_ref[0] = jnp.min(mmv_ref[0])
            mm_ref[1] = jnp.max(mmv_ref[1])
            mm_ref[2] = jnp.min(mmv_ref[2])
            mm_ref[3] = jnp.max(mmv_ref[3])

        mn1 = mm_ref[0]
        mx1 = mm_ref[1]
        mn2 = mm_ref[2]
        mx2 = mm_ref[3]
        inv1 = 1.0 / (mx1 - mn1 + _EPS)
        inv2 = 1.0 / (mx2 - mn2 + _EPS)

        rows = x1_ref.shape[1]
        dlt = _SQK / _NB
        for s in range(0, rows, 8):
            a = x1_ref[0, s:s + 8] * (0.5 * inv1)
            b = x2_ref[0, s:s + 8] * (0.5 * inv2)
            z = a + b - 0.5 * (mn1 * inv1 + mn2 * inv2)
            r = (a - b) * 2.0 + (mn2 * inv2 - mn1 * inv1)
            h = (r * r) * (-0.25 * _K2)
            zs = z * _SQK
            e = None
            d = None
            for bi in range(_NB):
                if bi % 16 == 0:
                    cbs = _SQK * ((bi + 0.5) / _NB)
                    zc = zs - cbs
                    e = h - zc * zc
                    d = zc * (2.0 * dlt) - dlt * dlt
                else:
                    e = e + d
                    d = d - 2.0 * dlt * dlt
                w = jnp.exp2(e)
                t = ((w[:, 0:128] + w[:, 128:256])
                     + (w[:, 256:384] + w[:, 384:512]))
                if s == 0:
                    hist_ref[i, bi] = t
                else:
                    hist_ref[i, bi] = hist_ref[i, bi] + t

        @pl.when(i == nc - 1)
        def _():
            total = 4 * 1 * 512 * 512
            part = jnp.sum(hist_ref[...], axis=2)
            hist = jnp.sum(part, axis=-1)
            hist = hist * (1.0 / total)
            ssum = jnp.sum(hist, axis=-1, keepdims=True)
            hist = hist / (ssum + _EPS)
            px = jnp.sum(hist, axis=-1, keepdims=True)
            py = jnp.sum(hist, axis=0, keepdims=True)
            px_py = px * py
            mi = jnp.sum(hist * jnp.log((hist + _EPS) / (px_py + _EPS) + _EPS))
            mi_ref[0] = mi


def kernel(img1, img2):
    n, c, h, w = img1.shape
    nc = n * c
    x1 = img1.reshape(nc, h, w)
    x2 = img2.reshape(nc, h, w)

    mi = pl.pallas_call(
        _body,
        grid=(2, nc),
        in_specs=[
            pl.BlockSpec((1, h, w), lambda p, i: (i, 0, 0)),
            pl.BlockSpec((1, h, w), lambda p, i: (i, 0, 0)),
        ],
        out_specs=pl.BlockSpec(memory_space=pltpu.SMEM),
        out_shape=jax.ShapeDtypeStruct((1,), jnp.float32),
        scratch_shapes=[
            pltpu.SMEM((8,), jnp.float32),
            pltpu.VMEM((nc, _NB, 8, 128), jnp.float32),
            pltpu.VMEM((4, 8, 128), jnp.float32),
        ],
        compiler_params=pltpu.CompilerParams(
            dimension_semantics=("arbitrary", "arbitrary"),
        ),
    )(x1, x2)
    return mi[0]


# speedup vs baseline: 3.7689x; 1.0018x over previous
import jax
import jax.numpy as jnp
import numpy as np
from jax.experimental import pallas as pl
from jax.experimental.pallas import tpu as pltpu

_NB = 64
_SIGMA = 0.1 * (1.0 / _NB)
_EPS = float(np.finfo(np.float32).eps)
_K2 = float(np.log2(np.e) / (_SIGMA * _SIGMA))
_SQK = float(np.sqrt(_K2))


def _fold_extreme(v, op):
    t = v[0:8]
    for s in range(8, v.shape[0], 8):
        t = op(t, v[s:s + 8])
    return op(op(t[:, 0:128], t[:, 128:256]),
              op(t[:, 256:384], t[:, 384:512]))


def _body(x1_ref, x2_ref, mi_ref, mm_ref, hist_ref, mmv_ref):
    p = pl.program_id(0)
    i = pl.program_id(1)
    nc = pl.num_programs(1)

    @pl.when(p == 0)
    def _():
        mn1 = _fold_extreme(x1_ref[0], jnp.minimum)
        mx1 = _fold_extreme(x1_ref[0], jnp.maximum)
        mn2 = _fold_extreme(x2_ref[0], jnp.minimum)
        mx2 = _fold_extreme(x2_ref[0], jnp.maximum)

        @pl.when(i == 0)
        def _():
            mmv_ref[0] = mn1
            mmv_ref[1] = mx1
            mmv_ref[2] = mn2
            mmv_ref[3] = mx2

        @pl.when(i > 0)
        def _():
            mmv_ref[0] = jnp.minimum(mmv_ref[0], mn1)
            mmv_ref[1] = jnp.maximum(mmv_ref[1], mx1)
            mmv_ref[2] = jnp.minimum(mmv_ref[2], mn2)
            mmv_ref[3] = jnp.maximum(mmv_ref[3], mx2)

    @pl.when(p == 1)
    def _():
        @pl.when(i == 0)
        def _():
            mm_ref[0] = jnp.min(mmv_ref[0])
            mm_ref[1] = jnp.max(mmv_ref[1])
            mm_ref[2] = jnp.min(mmv_ref[2])
            mm_ref[3] = jnp.max(mmv_ref[3])

        mn1 = mm_ref[0]
        mx1 = mm_ref[1]
        mn2 = mm_ref[2]
        mx2 = mm_ref[3]
        inv1 = 1.0 / (mx1 - mn1 + _EPS)
        inv2 = 1.0 / (mx2 - mn2 + _EPS)

        rows = x1_ref.shape[1]
        dlt = _SQK / _NB

        def _prep(s):
            a = x1_ref[0, s:s + 8] * (0.5 * inv1)
            b = x2_ref[0, s:s + 8] * (0.5 * inv2)
            z = a + b - 0.5 * (mn1 * inv1 + mn2 * inv2)
            r = (a - b) * 2.0 + (mn2 * inv2 - mn1 * inv1)
            h = (r * r) * (-0.25 * _K2)
            zs = z * _SQK
            return h, zs

        for s in range(0, rows, 16):
            hA, zsA = _prep(s)
            hB, zsB = _prep(s + 8)
            eA = dA = eB = dB = None
            for bi in range(_NB):
                if bi % 16 == 0:
                    cbs = _SQK * ((bi + 0.5) / _NB)
                    zcA = zsA - cbs
                    eA = hA - zcA * zcA
                    dA = zcA * (2.0 * dlt) - dlt * dlt
                    zcB = zsB - cbs
                    eB = hB - zcB * zcB
                    dB = zcB * (2.0 * dlt) - dlt * dlt
                else:
                    eA = eA + dA
                    dA = dA - 2.0 * dlt * dlt
                    eB = eB + dB
                    dB = dB - 2.0 * dlt * dlt
                wA = jnp.exp2(eA)
                wB = jnp.exp2(eB)
                t = (((wA[:, 0:128] + wA[:, 128:256])
                      + (wA[:, 256:384] + wA[:, 384:512]))
                     + ((wB[:, 0:128] + wB[:, 128:256])
                        + (wB[:, 256:384] + wB[:, 384:512])))
                if s == 0:
                    hist_ref[i, bi] = t
                else:
                    hist_ref[i, bi] = hist_ref[i, bi] + t

        @pl.when(i == nc - 1)
        def _():
            total = 4 * 1 * 512 * 512
            part = jnp.sum(hist_ref[...], axis=2)
            hist = jnp.sum(part, axis=-1)
            hist = hist * (1.0 / total)
            ssum = jnp.sum(hist, axis=-1, keepdims=True)
            hist = hist / (ssum + _EPS)
            px = jnp.sum(hist, axis=-1, keepdims=True)
            py = jnp.sum(hist, axis=0, keepdims=True)
            px_py = px * py
            mi = jnp.sum(hist * jnp.log((hist + _EPS) / (px_py + _EPS) + _EPS))
            mi_ref[0] = mi


def kernel(img1, img2):
    n, c, h, w = img1.shape
    nc = n * c
    x1 = img1.reshape(nc, h, w)
    x2 = img2.reshape(nc, h, w)

    mi = pl.pallas_call(
        _body,
        grid=(2, nc),
        in_specs=[
            pl.BlockSpec((1, h, w), lambda p, i: (i, 0, 0)),
            pl.BlockSpec((1, h, w), lambda p, i: (i, 0, 0)),
        ],
        out_specs=pl.BlockSpec(memory_space=pltpu.SMEM),
        out_shape=jax.ShapeDtypeStruct((1,), jnp.float32),
        scratch_shapes=[
            pltpu.SMEM((8,), jnp.float32),
            pltpu.VMEM((nc, _NB, 8, 128), jnp.float32),
            pltpu.VMEM((4, 8, 128), jnp.float32),
        ],
        compiler_params=pltpu.CompilerParams(
            dimension_semantics=("arbitrary", "arbitrary"),
        ),
    )(x1, x2)
    return mi[0]
